# baseline (speedup 1.0000x reference)
import functools

import jax
import jax.numpy as jnp
from jax import lax
from jax.experimental import pallas as pl
from jax.experimental.pallas import tpu as pltpu
from jax.experimental.pallas import tpu_sc as plsc

N_FACTORS = 32
BATCH = 16384
N_ROWS = 1000000
N_COLS = 7813
NC = 2
NS = 16
NW = NC * NS
RANGE = 245
SLAB_COLS = 8
SLAB_W = SLAB_COLS * 128
NSLAB = 31
MAX_FETCH_COL = N_COLS - SLAB_COLS
NVEC = BATCH // 16
STAGE = 64
UROWS = BATCH + STAGE
BPW = BATCH // NW
CH = 256

_mesh = plsc.VectorSubcoreMesh(core_axis_name="c", subcore_axis_name="s")


@functools.partial(
    pl.kernel,
    out_type=(
        jax.ShapeDtypeStruct((UROWS, 128), jnp.float32),
        jax.ShapeDtypeStruct((UROWS, 128), jnp.float32),
    ),
    mesh=_mesh,
    compiler_params=pltpu.CompilerParams(
        needs_layout_passes=False, use_tc_tiling_on_sc=True
    ),
    scratch_types=[
        pltpu.VMEM((BATCH,), jnp.int32),
        pltpu.VMEM((BATCH + 16,), jnp.int32),
        pltpu.VMEM((N_FACTORS, 2 * SLAB_W), jnp.float32),
        pltpu.VMEM((STAGE, 128), jnp.float32),
        pltpu.VMEM((STAGE + 16,), jnp.int32),
        pltpu.VMEM((32,), jnp.int32),
        pltpu.SMEM((4,), jnp.int32),
        pltpu.SemaphoreType.DMA,
        pltpu.SemaphoreType.DMA,
        pltpu.SemaphoreType.DMA,
    ],
)
def _sc_gather_kernel(users_hbm, items_hbm, ut_hbm, vt_hbm, u_out, v_out,
                      allidx, mlist, slabs, stag, sids, tmp, curs,
                      sem_a, sem_b, sem_f):
    wid = lax.axis_index("s") * NC + lax.axis_index("c")
    tc_lo = wid * RANGE
    lo = tc_lo * 128
    hi = jnp.minimum((tc_lo + RANGE) * 128, N_ROWS)
    lane = lax.iota(jnp.int32, 16)
    lane0 = lane == 0

    def reset_sids():
        for t in range(STAGE // 16):
            sids[pl.ds(t * 16, 16)] = BATCH + t * 16 + lane

    def flush(table_out):
        pltpu.async_copy(stag, table_out.at[sids.at[pl.ds(0, STAGE)]],
                         sem_f).wait()
        reset_sids()
        curs[0] = 0

    def emit_table(idx_hbm, table_hbm, table_out):
        pltpu.sync_copy(idx_hbm, allidx)
        reset_sids()
        curs[0] = 0

        def scan(k, cnt):
            vec = allidx[pl.ds(k * 16, 16)]
            m = (vec >= lo) & (vec < hi)
            tcrel = (vec >> 7) - tc_lo
            entry = (tcrel << 21) | ((vec & 127) << 14) | (k * 16 + lane)
            plsc.store_compressed(mlist.at[pl.ds(cnt, 16)], entry, mask=m)
            pc = plsc.all_reduce_population_count(m)
            return cnt + pc[0]

        cnt = lax.fori_loop(0, NVEC, scan, 0)
        nvec = (cnt + 15) >> 4

        def issue_slab(s, slot):
            col0 = jnp.minimum(tc_lo + s * SLAB_COLS, MAX_FETCH_COL)
            off = pl.multiple_of(col0 * 128, 128)
            pltpu.async_copy(
                table_hbm.at[:, pl.ds(off, SLAB_W)],
                slabs.at[:, pl.ds(slot * SLAB_W, SLAB_W)],
                sem_a if slot == 0 else sem_b)

        def wait_slab(slot):
            pltpu.make_async_copy(
                table_hbm.at[:, pl.ds(0, SLAB_W)],
                slabs.at[:, pl.ds(slot * SLAB_W, SLAB_W)],
                sem_a if slot == 0 else sem_b).wait()

        def process_slab(s, slot):
            col0 = jnp.minimum(tc_lo + s * SLAB_COLS, MAX_FETCH_COL)
            sbase = slot * SLAB_W

            def mvec_body(k, _):
                ev = mlist[pl.ds(k * 16, 16)]
                valid = (k * 16 + lane) < cnt
                m = ((ev >> 24) == s) & valid
                pc = plsc.all_reduce_population_count(m)[0]

                @pl.when(pc > 0)
                def _():
                    plsc.store_compressed(tmp.at[pl.ds(0, 16)], ev, mask=m)

                    def member(j, sc):
                        e = tmp[pl.ds(j, 16)][0]
                        tc = tc_lo + (e >> 21)
                        ln = (e >> 14) & 127
                        eid = e & 16383
                        col = sbase + (tc - col0) * 128 + ln
                        cvec = jnp.full((16,), col, jnp.int32)
                        x_lo = plsc.load_gather(slabs, [lane, cvec])
                        x_hi = plsc.load_gather(slabs, [lane + 16, cvec])
                        scv = jnp.full((16,), sc, jnp.int32)
                        plsc.store_scatter(stag, [scv, lane], x_lo)
                        plsc.store_scatter(stag, [scv, lane + 16], x_hi)
                        plsc.store_scatter(sids, [scv],
                                           jnp.full((16,), eid, jnp.int32),
                                           mask=lane0)

                        @pl.when(sc + 1 == STAGE)
                        def _():
                            flush(table_out)

                        return jnp.where(sc + 1 == STAGE, 0, sc + 1)

                    sc_end = lax.fori_loop(0, pc, member, curs[0])
                    curs[0] = sc_end

                return 0

            lax.fori_loop(0, nvec, mvec_body, 0)

        issue_slab(0, 0)
        for i in range(NSLAB // 2):
            issue_slab(2 * i + 1, 1)
            wait_slab(0)
            process_slab(2 * i, 0)
            issue_slab(2 * i + 2, 0)
            wait_slab(1)
            process_slab(2 * i + 1, 1)
        wait_slab(0)
        process_slab(NSLAB - 1, 0)
        flush(table_out)

    emit_table(users_hbm, ut_hbm, u_out)
    emit_table(items_hbm, vt_hbm, v_out)


@functools.partial(
    pl.kernel,
    out_type=jax.ShapeDtypeStruct((BATCH,), jnp.float32),
    mesh=_mesh,
    compiler_params=pltpu.CompilerParams(
        needs_layout_passes=False, use_tc_tiling_on_sc=True
    ),
    scratch_types=[
        pltpu.VMEM((CH, 128), jnp.float32),
        pltpu.VMEM((CH, 128), jnp.float32),
        pltpu.VMEM((BPW,), jnp.float32),
    ],
)
def _sc_dot_kernel(u_hbm, v_hbm, out_hbm, ubuf, vbuf, outv):
    wid = lax.axis_index("s") * NC + lax.axis_index("c")
    base = wid * BPW
    lane = lax.iota(jnp.int32, 16)
    for c in range(BPW // CH):
        pltpu.sync_copy(u_hbm.at[pl.ds(base + c * CH, CH)], ubuf)
        pltpu.sync_copy(v_hbm.at[pl.ds(base + c * CH, CH)], vbuf)

        def g_body(g, _, _c=c):
            rows = g * 16 + lane
            acc = jnp.zeros((16,), jnp.float32)
            for d in range(N_FACTORS):
                dv = jnp.full((16,), d, jnp.int32)
                acc = acc + (plsc.load_gather(ubuf, [rows, dv]) *
                             plsc.load_gather(vbuf, [rows, dv]))
            outv[pl.ds(_c * CH + g * 16, 16)] = acc
            return 0

        lax.fori_loop(0, CH // 16, g_body, 0)
    pltpu.sync_copy(outv, out_hbm.at[pl.ds(base, BPW)])


def kernel(data, user_factors, item_factors):
    users = data[:, 0].astype(jnp.int32)
    items = data[:, 1].astype(jnp.int32)
    u16, v16 = _sc_gather_kernel(users, items,
                                 user_factors.T, item_factors.T)
    return _sc_dot_kernel(u16, v16)


# speedup vs baseline: 1.0077x; 1.0077x over previous
import functools

import jax
import jax.numpy as jnp
from jax import lax
from jax.experimental import pallas as pl
from jax.experimental.pallas import tpu as pltpu
from jax.experimental.pallas import tpu_sc as plsc

N_FACTORS = 32
BATCH = 16384
N_ROWS = 1000000
N_COLS = 7813
NC = 2
NS = 16
NW = NC * NS
RANGE = 245
SLAB_COLS = 8
SLAB_W = SLAB_COLS * 128
NSLAB = 31
MAX_FETCH_COL = N_COLS - SLAB_COLS
NVEC = BATCH // 16
STAGE = 64
UROWS = BATCH + STAGE
BPW = BATCH // NW
CH = 256

_mesh = plsc.VectorSubcoreMesh(core_axis_name="c", subcore_axis_name="s")


@functools.partial(
    pl.kernel,
    out_type=(
        jax.ShapeDtypeStruct((UROWS, 128), jnp.float32),
        jax.ShapeDtypeStruct((UROWS, 128), jnp.float32),
    ),
    mesh=_mesh,
    compiler_params=pltpu.CompilerParams(
        needs_layout_passes=False, use_tc_tiling_on_sc=True
    ),
    scratch_types=[
        pltpu.VMEM((BATCH,), jnp.int32),
        pltpu.VMEM((BATCH + 16,), jnp.int32),
        pltpu.VMEM((N_FACTORS, 2 * SLAB_W), jnp.float32),
        pltpu.VMEM((STAGE, 128), jnp.float32),
        pltpu.VMEM((STAGE + 16,), jnp.int32),
        pltpu.VMEM((32,), jnp.int32),
        pltpu.SMEM((4,), jnp.int32),
        pltpu.SemaphoreType.DMA,
        pltpu.SemaphoreType.DMA,
        pltpu.SemaphoreType.DMA,
    ],
)
def _sc_gather_kernel(users_hbm, items_hbm, ut_hbm, vt_hbm, u_out, v_out,
                      allidx, mlist, slabs, stag, sids, tmp, curs,
                      sem_a, sem_b, sem_f):
    wid = lax.axis_index("s") * NC + lax.axis_index("c")
    tc_lo = wid * RANGE
    lo = tc_lo * 128
    hi = jnp.minimum((tc_lo + RANGE) * 128, N_ROWS)
    lane = lax.iota(jnp.int32, 16)
    lane0 = lane == 0

    def reset_sids():
        for t in range(STAGE // 16):
            sids[pl.ds(t * 16, 16)] = BATCH + t * 16 + lane

    def flush(table_out):
        pltpu.async_copy(stag, table_out.at[sids.at[pl.ds(0, STAGE)]],
                         sem_f).wait()
        reset_sids()
        curs[0] = 0

    def emit_table(idx_hbm, table_hbm, table_out):
        pltpu.sync_copy(idx_hbm, allidx)
        reset_sids()
        curs[0] = 0

        def scan(k, cnt):
            for u in range(2):
                vec = allidx[pl.ds((2 * k + u) * 16, 16)]
                m = (vec >= lo) & (vec < hi)
                tcrel = (vec >> 7) - tc_lo
                entry = ((tcrel << 21) | ((vec & 127) << 14)
                         | ((2 * k + u) * 16 + lane))
                plsc.store_compressed(mlist.at[pl.ds(cnt, 16)], entry,
                                      mask=m)
                cnt = cnt + plsc.all_reduce_population_count(m)[0]
            return cnt

        cnt = lax.fori_loop(0, NVEC // 2, scan, 0)
        nvec = (cnt + 15) >> 4

        def issue_slab(s, slot):
            col0 = jnp.minimum(tc_lo + s * SLAB_COLS, MAX_FETCH_COL)
            off = pl.multiple_of(col0 * 128, 128)
            pltpu.async_copy(
                table_hbm.at[:, pl.ds(off, SLAB_W)],
                slabs.at[:, pl.ds(slot * SLAB_W, SLAB_W)],
                sem_a if slot == 0 else sem_b)

        def wait_slab(slot):
            pltpu.make_async_copy(
                table_hbm.at[:, pl.ds(0, SLAB_W)],
                slabs.at[:, pl.ds(slot * SLAB_W, SLAB_W)],
                sem_a if slot == 0 else sem_b).wait()

        def process_slab(s, slot):
            col0 = jnp.minimum(tc_lo + s * SLAB_COLS, MAX_FETCH_COL)
            sbase = slot * SLAB_W

            def mvec_body(k, _):
                ev = mlist[pl.ds(k * 16, 16)]
                valid = (k * 16 + lane) < cnt
                m = ((ev >> 24) == s) & valid
                pc = plsc.all_reduce_population_count(m)[0]

                @pl.when(pc > 0)
                def _():
                    plsc.store_compressed(tmp.at[pl.ds(0, 16)], ev, mask=m)

                    def member(j, sc):
                        e = tmp[pl.ds(j, 16)][0]
                        tc = tc_lo + (e >> 21)
                        ln = (e >> 14) & 127
                        eid = e & 16383
                        col = sbase + (tc - col0) * 128 + ln
                        cvec = jnp.full((16,), col, jnp.int32)
                        x_lo = plsc.load_gather(slabs, [lane, cvec])
                        x_hi = plsc.load_gather(slabs, [lane + 16, cvec])
                        scv = jnp.full((16,), sc, jnp.int32)
                        plsc.store_scatter(stag, [scv, lane], x_lo)
                        plsc.store_scatter(stag, [scv, lane + 16], x_hi)
                        plsc.store_scatter(sids, [scv],
                                           jnp.full((16,), eid, jnp.int32),
                                           mask=lane0)

                        @pl.when(sc + 1 == STAGE)
                        def _():
                            flush(table_out)

                        return jnp.where(sc + 1 == STAGE, 0, sc + 1)

                    sc_end = lax.fori_loop(0, pc, member, curs[0])
                    curs[0] = sc_end

                return 0

            lax.fori_loop(0, nvec, mvec_body, 0)

        issue_slab(0, 0)
        for i in range(NSLAB // 2):
            issue_slab(2 * i + 1, 1)
            wait_slab(0)
            process_slab(2 * i, 0)
            issue_slab(2 * i + 2, 0)
            wait_slab(1)
            process_slab(2 * i + 1, 1)
        wait_slab(0)
        process_slab(NSLAB - 1, 0)
        flush(table_out)

    emit_table(users_hbm, ut_hbm, u_out)
    emit_table(items_hbm, vt_hbm, v_out)


@functools.partial(
    pl.kernel,
    out_type=jax.ShapeDtypeStruct((BATCH,), jnp.float32),
    mesh=_mesh,
    compiler_params=pltpu.CompilerParams(
        needs_layout_passes=False, use_tc_tiling_on_sc=True
    ),
    scratch_types=[
        pltpu.VMEM((CH, 128), jnp.float32),
        pltpu.VMEM((CH, 128), jnp.float32),
        pltpu.VMEM((BPW,), jnp.float32),
    ],
)
def _sc_dot_kernel(u_hbm, v_hbm, out_hbm, ubuf, vbuf, outv):
    wid = lax.axis_index("s") * NC + lax.axis_index("c")
    base = wid * BPW
    lane = lax.iota(jnp.int32, 16)
    d_consts = [jnp.full((16,), d, jnp.int32) for d in range(N_FACTORS)]
    for c in range(BPW // CH):
        pltpu.sync_copy(u_hbm.at[pl.ds(base + c * CH, CH)], ubuf)
        pltpu.sync_copy(v_hbm.at[pl.ds(base + c * CH, CH)], vbuf)

        def g_body(g, _, _c=c):
            rows = g * 16 + lane
            acc = jnp.zeros((16,), jnp.float32)
            for d in range(N_FACTORS):
                acc = acc + (plsc.load_gather(ubuf, [rows, d_consts[d]]) *
                             plsc.load_gather(vbuf, [rows, d_consts[d]]))
            outv[pl.ds(_c * CH + g * 16, 16)] = acc
            return 0

        lax.fori_loop(0, CH // 16, g_body, 0)
    pltpu.sync_copy(outv, out_hbm.at[pl.ds(base, BPW)])


def kernel(data, user_factors, item_factors):
    users = data[:, 0].astype(jnp.int32)
    items = data[:, 1].astype(jnp.int32)
    u16, v16 = _sc_gather_kernel(users, items,
                                 user_factors.T, item_factors.T)
    return _sc_dot_kernel(u16, v16)
